# Initial kernel scaffold; baseline (speedup 1.0000x reference)
import jax
import jax.numpy as jnp
from jax.experimental import pallas as pl


def kernel(input_ids, edit_type_ids, word_emb, pos_emb, edit_emb, gamma, beta):
    raise NotImplementedError("write your pallas kernel here")


# speedup vs baseline: 1.5089x; 1.5089x over previous
import functools

import jax
import jax.numpy as jnp
from jax import lax
from jax.experimental import pallas as pl
from jax.experimental.pallas import tpu as pltpu
from jax.experimental.pallas import tpu_sc as plsc

VOCAB = 100000
HIDDEN = 768
N_EDIT = 5
EPS = 1e-12

NUM_CORES = 2
NUM_SUBCORES = 16
NUM_WORKERS = NUM_CORES * NUM_SUBCORES
CHUNK = 128

TC_BLOCK = 256


def _sc_gather(idx, table, n_tokens):
    per_worker = n_tokens // NUM_WORKERS
    n_chunks = per_worker // CHUNK

    @functools.partial(
        pl.kernel,
        out_type=jax.ShapeDtypeStruct((n_tokens, HIDDEN), table.dtype),
        mesh=plsc.VectorSubcoreMesh(core_axis_name="c", subcore_axis_name="s"),
        scratch_types=[
            pltpu.VMEM((CHUNK,), jnp.int32),
            pltpu.VMEM((CHUNK, HIDDEN), table.dtype),
            pltpu.SemaphoreType.DMA,
        ],
    )
    def gather_kernel(idx_hbm, table_hbm, out_hbm, idx_v, rows_v, sem):
        wid = lax.axis_index("s") * NUM_CORES + lax.axis_index("c")
        base = wid * per_worker

        @pl.loop(0, n_chunks)
        def _(c):
            off = pl.multiple_of(base + c * CHUNK, CHUNK)
            pltpu.sync_copy(idx_hbm.at[pl.ds(off, CHUNK)], idx_v)
            pltpu.async_copy(table_hbm.at[idx_v], rows_v, sem).wait()
            pltpu.sync_copy(rows_v, out_hbm.at[pl.ds(off, CHUNK)])

    return gather_kernel(idx, table)


def _tc_finish_body(rows_ref, pos_ref, eids_ref, edit_ref, gamma_ref, beta_ref,
                    out_ref):
    x = rows_ref[...] + pos_ref[...]
    eids = eids_ref[...]
    for k in range(N_EDIT):
        x = x + jnp.where(eids == k, edit_ref[k:k + 1, :], 0.0)
    mean = jnp.mean(x, axis=1, keepdims=True)
    d = x - mean
    var = jnp.mean(d * d, axis=1, keepdims=True)
    xhat = d * lax.rsqrt(var + EPS)
    out_ref[...] = xhat * gamma_ref[...] + beta_ref[...]


def _tc_finish(rows, pos_emb, eids, edit_emb, gamma, beta, n_tokens, seq_len):
    grid = n_tokens // TC_BLOCK
    blocks_per_batch = seq_len // TC_BLOCK
    edit_pad = jnp.zeros((8, HIDDEN), edit_emb.dtype).at[:N_EDIT].set(edit_emb)
    return pl.pallas_call(
        _tc_finish_body,
        grid=(grid,),
        in_specs=[
            pl.BlockSpec((TC_BLOCK, HIDDEN), lambda i: (i, 0)),
            pl.BlockSpec((TC_BLOCK, HIDDEN),
                         lambda i: (i % blocks_per_batch, 0)),
            pl.BlockSpec((TC_BLOCK, 1), lambda i: (i, 0)),
            pl.BlockSpec((8, HIDDEN), lambda i: (0, 0)),
            pl.BlockSpec((1, HIDDEN), lambda i: (0, 0)),
            pl.BlockSpec((1, HIDDEN), lambda i: (0, 0)),
        ],
        out_specs=pl.BlockSpec((TC_BLOCK, HIDDEN), lambda i: (i, 0)),
        out_shape=jax.ShapeDtypeStruct((n_tokens, HIDDEN), rows.dtype),
    )(rows, pos_emb, eids, edit_pad, gamma.reshape(1, HIDDEN),
      beta.reshape(1, HIDDEN))


def kernel(input_ids, edit_type_ids, word_emb, pos_emb, edit_emb, gamma, beta):
    b, s = input_ids.shape
    n_tokens = b * s
    idx = input_ids.reshape(n_tokens).astype(jnp.int32)
    eids = edit_type_ids.reshape(n_tokens, 1).astype(jnp.int32)
    rows = _sc_gather(idx, word_emb, n_tokens)
    out = _tc_finish(rows, pos_emb, eids, edit_emb, gamma, beta, n_tokens, s)
    return out.reshape(b, s, HIDDEN)
